# Initial kernel scaffold; baseline (speedup 1.0000x reference)
import jax
import jax.numpy as jnp
from jax.experimental import pallas as pl


def kernel(feat, edge_index, Wg, bg):
    raise NotImplementedError("write your pallas kernel here")


# speedup vs baseline: 4.3967x; 4.3967x over previous
import functools

import jax
import jax.numpy as jnp
from jax import lax
from jax.experimental import pallas as pl
from jax.experimental.pallas import tpu as pltpu
from jax.experimental.pallas import tpu_sc as plsc

K = 10
N = 10000
E = 320000
D = 128
ALPHA = 0.1

NC = 2
NS = 16
CHUNK = 80
EDGES_PER_WORKER = E // (NC * NS)
N_CHUNKS = EDGES_PER_WORKER // CHUNK
NPAD = 10240
ROWS_PER_SUB = NPAD // NS
STAGE_ROWS = 128

_f32 = jnp.float32


def _gate_math(h, wg, bg):
    logits = jnp.dot(h, wg, preferred_element_type=_f32) + bg[0, 0]
    m = jnp.max(logits)
    e = jnp.exp(logits - m)
    gate = e / jnp.sum(e)
    return h * gate


def _gate_body(h_ref, wg_ref, bg_ref, r_ref):
    r_ref[...] = _gate_math(h_ref[...], wg_ref[...], bg_ref[...])


def _blend_gate_body(parts_ref, feat0_ref, wg_ref, bg_ref, h_ref, r_ref):
    neigh = parts_ref[0:N, :] + parts_ref[NPAD:NPAD + N, :]
    h = (1.0 - ALPHA) * neigh + ALPHA * feat0_ref[...]
    h_ref[...] = h
    r_ref[...] = _gate_math(h, wg_ref[...], bg_ref[...])


def _blend_body(parts_ref, feat0_ref, h_ref):
    neigh = parts_ref[0:N, :] + parts_ref[NPAD:NPAD + N, :]
    h_ref[...] = (1.0 - ALPHA) * neigh + ALPHA * feat0_ref[...]


_nd = jax.ShapeDtypeStruct((N, D), _f32)

_tc_gate = pl.pallas_call(_gate_body, out_shape=_nd)
_tc_blend_gate = pl.pallas_call(_blend_gate_body, out_shape=(_nd, _nd))
_tc_blend = pl.pallas_call(_blend_body, out_shape=_nd)


def _sc_scatter_body(r_hbm, src_hbm, dst_hbm, out_hbm,
                     accum, src_v, dst_v, rows_v, stage, sem):
    c = lax.axis_index("c")
    s = lax.axis_index("s")
    w = s * NC + c

    def _zero_body(i, carry):
        stage[i // 8, pl.ds((i % 8) * 16, 16)] = jnp.zeros((16,), _f32)
        return carry

    lax.fori_loop(0, STAGE_ROWS * 8, _zero_body, 0)
    for t in range(ROWS_PER_SUB // STAGE_ROWS):
        row0 = s * ROWS_PER_SUB + t * STAGE_ROWS
        pltpu.sync_copy(stage, accum.at[pl.ds(row0, STAGE_ROWS)])
    plsc.subcore_barrier()

    def _edge_body(j, carry):
        base = pl.multiple_of(w * EDGES_PER_WORKER + j * CHUNK, 8)
        pltpu.sync_copy(src_hbm.at[pl.ds(base, CHUNK)], src_v)
        pltpu.sync_copy(dst_hbm.at[pl.ds(base, CHUNK)], dst_v)
        pltpu.async_copy(r_hbm.at[src_v], rows_v, sem).wait()
        pltpu.sync_copy(rows_v, accum.at[dst_v], add=True)
        return carry

    lax.fori_loop(0, N_CHUNKS, _edge_body, 0)
    plsc.subcore_barrier()

    row0 = s * ROWS_PER_SUB
    pltpu.sync_copy(accum.at[pl.ds(row0, ROWS_PER_SUB)],
                    out_hbm.at[pl.ds(c * NPAD + row0, ROWS_PER_SUB)])


@functools.cache
def _get_sc_scatter():
    return pl.kernel(
        _sc_scatter_body,
        out_type=jax.ShapeDtypeStruct((2 * NPAD, D), _f32),
        mesh=plsc.VectorSubcoreMesh(core_axis_name="c", subcore_axis_name="s"),
        scratch_types=[
            pltpu.VMEM_SHARED((NPAD, D), _f32),
            pltpu.VMEM((CHUNK,), jnp.int32),
            pltpu.VMEM((CHUNK,), jnp.int32),
            pltpu.VMEM((CHUNK, D), _f32),
            pltpu.VMEM((STAGE_ROWS, D), _f32),
            pltpu.SemaphoreType.DMA,
        ],
    )


@jax.jit
def kernel(feat, edge_index, Wg, bg):
    src = edge_index[0]
    dst = edge_index[1]
    sc_scatter = _get_sc_scatter()
    feats = []
    r = _tc_gate(feat, Wg[0], bg[0].reshape(1, 1))
    for i in range(K):
        parts = sc_scatter(r, src, dst)
        if i < K - 1:
            h, r = _tc_blend_gate(parts, feat, Wg[i + 1], bg[i + 1].reshape(1, 1))
        else:
            h = _tc_blend(parts, feat)
        feats.append(h)
    return jnp.stack(feats, axis=0)


# speedup vs baseline: 12.1040x; 2.7530x over previous
import functools

import jax
import jax.numpy as jnp
from jax import lax
from jax.experimental import pallas as pl
from jax.experimental.pallas import tpu as pltpu
from jax.experimental.pallas import tpu_sc as plsc

K = 10
N = 10000
E = 320000
D = 128
ALPHA = 0.1

NC = 2
NS = 16
CHUNK = 40
EDGES_PER_WORKER = E // (NC * NS)
N_CHUNKS = EDGES_PER_WORKER // CHUNK
NPAD = 10240
ROWS_PER_SUB = NPAD // NS

_f32 = jnp.float32


def _gate_math(h, wg, bg):
    logits = jnp.dot(h, wg, preferred_element_type=_f32) + bg[0, 0]
    m = jnp.max(logits)
    e = jnp.exp(logits - m)
    gate = e / jnp.sum(e)
    return h * gate


def _gate_body(h_ref, wg_ref, bg_ref, r_ref):
    r_ref[...] = _gate_math(h_ref[...], wg_ref[...], bg_ref[...])


def _blend_gate_body(parts_ref, feat0_ref, wg_ref, bg_ref, h_ref, r_ref):
    neigh = parts_ref[0:N, :] + parts_ref[NPAD:NPAD + N, :]
    h = (1.0 - ALPHA) * neigh + ALPHA * feat0_ref[...]
    h_ref[...] = h
    r_ref[...] = _gate_math(h, wg_ref[...], bg_ref[...])


def _blend_body(parts_ref, feat0_ref, h_ref):
    neigh = parts_ref[0:N, :] + parts_ref[NPAD:NPAD + N, :]
    h_ref[...] = (1.0 - ALPHA) * neigh + ALPHA * feat0_ref[...]


_nd = jax.ShapeDtypeStruct((N, D), _f32)

_tc_gate = pl.pallas_call(_gate_body, out_shape=_nd)
_tc_blend_gate = pl.pallas_call(_blend_gate_body, out_shape=(_nd, _nd))
_tc_blend = pl.pallas_call(_blend_body, out_shape=_nd)


NBUF = 5


def _sc_scatter_body(r_hbm, src_hbm, dst_hbm, out_hbm,
                     accum, src_buf,
                     dmini0, dmini1, dmini2, dmini3, dmini4,
                     rows0, rows1, rows2, rows3, rows4,
                     zsem,
                     isem0, isem1, isem2, isem3, isem4,
                     gsem0, gsem1, gsem2, gsem3, gsem4,
                     ssem0, ssem1, ssem2, ssem3, ssem4):
    dmini = [dmini0, dmini1, dmini2, dmini3, dmini4]
    rows = [rows0, rows1, rows2, rows3, rows4]
    isems = [isem0, isem1, isem2, isem3, isem4]
    gsems = [gsem0, gsem1, gsem2, gsem3, gsem4]
    ssems = [ssem0, ssem1, ssem2, ssem3, ssem4]
    c = lax.axis_index("c")
    s = lax.axis_index("s")
    w = s * NC + c
    ebase = pl.multiple_of(w * EDGES_PER_WORKER, 8)

    icp_s = pltpu.async_copy(
        src_hbm.at[pl.ds(ebase, EDGES_PER_WORKER)], src_buf, zsem)

    def _zero_body(i, carry):
        rows0[i // 8, pl.ds((i % 8) * 16, 16)] = jnp.zeros((16,), _f32)
        return carry

    lax.fori_loop(0, CHUNK * 8, _zero_body, 0)
    zcps = []
    for t in range(ROWS_PER_SUB // CHUNK):
        row0 = s * ROWS_PER_SUB + t * CHUNK
        zcps.append(pltpu.async_copy(rows0, accum.at[pl.ds(row0, CHUNK)], zsem))
    icp_s.wait()
    for cp in zcps:
        cp.wait()
    plsc.subcore_barrier()

    def _turn(g, b):
        b1 = (b - 1) % NBUF
        b3 = (b - 3) % NBUF

        @pl.when(g >= NBUF)
        def _():
            pltpu.make_async_copy(
                rows[b], accum.at[dmini[b]], ssems[b]).wait()

        @pl.when(g < N_CHUNKS)
        def _():
            pltpu.async_copy(
                dst_hbm.at[pl.ds(pl.multiple_of(ebase + g * CHUNK, 8), CHUNK)],
                dmini[b], isems[b])

        g1 = g - 1

        @pl.when((g1 >= 0) & (g1 < N_CHUNKS))
        def _():
            idx = src_buf.at[pl.ds(pl.multiple_of(g1 * CHUNK, 8), CHUNK)]
            pltpu.async_copy(r_hbm.at[idx], rows[b1], gsems[b1])

        g3 = g - 3

        @pl.when((g3 >= 0) & (g3 < N_CHUNKS))
        def _():
            pltpu.make_async_copy(
                dst_hbm.at[pl.ds(ebase, CHUNK)], dmini[b3], isems[b3]).wait()
            idx = src_buf.at[pl.ds(0, CHUNK)]
            pltpu.make_async_copy(r_hbm.at[idx], rows[b3], gsems[b3]).wait()
            pltpu.async_copy(rows[b3], accum.at[dmini[b3]], ssems[b3],
                             add=True)

    def _outer(o, carry):
        for b in range(NBUF):
            _turn(o * NBUF + b, b)
        return carry

    n_turns = N_CHUNKS + NBUF
    lax.fori_loop(0, n_turns // NBUF, _outer, 0)
    plsc.subcore_barrier()

    row0 = s * ROWS_PER_SUB
    pltpu.sync_copy(accum.at[pl.ds(row0, ROWS_PER_SUB)],
                    out_hbm.at[pl.ds(c * NPAD + row0, ROWS_PER_SUB)])


@functools.cache
def _get_sc_scatter():
    return pl.kernel(
        _sc_scatter_body,
        out_type=jax.ShapeDtypeStruct((2 * NPAD, D), _f32),
        mesh=plsc.VectorSubcoreMesh(core_axis_name="c", subcore_axis_name="s"),
        scratch_types=[
            pltpu.VMEM_SHARED((NPAD, D), _f32),
            pltpu.VMEM((EDGES_PER_WORKER,), jnp.int32),
            *[pltpu.VMEM((CHUNK,), jnp.int32) for _ in range(NBUF)],
            *[pltpu.VMEM((CHUNK, D), _f32) for _ in range(NBUF)],
            *[pltpu.SemaphoreType.DMA for _ in range(3 * NBUF + 1)],
        ],
    )


@jax.jit
def kernel(feat, edge_index, Wg, bg):
    src = edge_index[0]
    dst = edge_index[1]
    sc_scatter = _get_sc_scatter()
    feats = []
    r = _tc_gate(feat, Wg[0], bg[0].reshape(1, 1))
    for i in range(K):
        parts = sc_scatter(r, src, dst)
        if i < K - 1:
            h, r = _tc_blend_gate(parts, feat, Wg[i + 1], bg[i + 1].reshape(1, 1))
        else:
            h = _tc_blend(parts, feat)
        feats.append(h)
    return jnp.stack(feats, axis=0)


# speedup vs baseline: 13.0915x; 1.0816x over previous
import functools

import jax
import jax.numpy as jnp
from jax import lax
from jax.experimental import pallas as pl
from jax.experimental.pallas import tpu as pltpu
from jax.experimental.pallas import tpu_sc as plsc

K = 10
N = 10000
E = 320000
D = 128
ALPHA = 0.1

NC = 2
NS = 16
CHUNK = 80
EDGES_PER_WORKER = E // (NC * NS)
N_CHUNKS = EDGES_PER_WORKER // CHUNK
NPAD = 10240
ROWS_PER_SUB = NPAD // NS

_f32 = jnp.float32


def _gate_math(h, wg, bg):
    logits = jnp.dot(h, wg, preferred_element_type=_f32) + bg[0, 0]
    m = jnp.max(logits)
    e = jnp.exp(logits - m)
    gate = e / jnp.sum(e)
    return h * gate


def _gate_body(h_ref, wg_ref, bg_ref, r_ref):
    r_ref[...] = _gate_math(h_ref[...], wg_ref[...], bg_ref[...])


def _blend_gate_body(parts_ref, feat0_ref, wg_ref, bg_ref, h_ref, r_ref):
    neigh = parts_ref[0:N, :] + parts_ref[NPAD:NPAD + N, :]
    h = (1.0 - ALPHA) * neigh + ALPHA * feat0_ref[...]
    h_ref[...] = h
    r_ref[...] = _gate_math(h, wg_ref[...], bg_ref[...])


def _blend_body(parts_ref, feat0_ref, h_ref):
    neigh = parts_ref[0:N, :] + parts_ref[NPAD:NPAD + N, :]
    h_ref[...] = (1.0 - ALPHA) * neigh + ALPHA * feat0_ref[...]


_nd = jax.ShapeDtypeStruct((N, D), _f32)

_tc_gate = pl.pallas_call(_gate_body, out_shape=_nd)
_tc_blend_gate = pl.pallas_call(_blend_gate_body, out_shape=(_nd, _nd))
_tc_blend = pl.pallas_call(_blend_body, out_shape=_nd)


NBUF = 4


def _sc_scatter_body(r_hbm, src_hbm, dst_hbm, out_hbm,
                     accum,
                     smini0, smini1, smini2, smini3,
                     dmini0, dmini1, dmini2, dmini3,
                     rows0, rows1, rows2, rows3,
                     zsem,
                     isem0, isem1, isem2, isem3,
                     gsem0, gsem1, gsem2, gsem3,
                     ssem0, ssem1, ssem2, ssem3):
    smini = [smini0, smini1, smini2, smini3]
    dmini = [dmini0, dmini1, dmini2, dmini3]
    rows = [rows0, rows1, rows2, rows3]
    isems = [isem0, isem1, isem2, isem3]
    gsems = [gsem0, gsem1, gsem2, gsem3]
    ssems = [ssem0, ssem1, ssem2, ssem3]
    c = lax.axis_index("c")
    s = lax.axis_index("s")
    w = s * NC + c
    ebase = pl.multiple_of(w * EDGES_PER_WORKER, 8)

    def _zero_body(i, carry):
        rows0[i // 8, pl.ds((i % 8) * 16, 16)] = jnp.zeros((16,), _f32)
        return carry

    lax.fori_loop(0, CHUNK * 8, _zero_body, 0)
    zcps = []
    for t in range(ROWS_PER_SUB // CHUNK):
        row0 = s * ROWS_PER_SUB + t * CHUNK
        zcps.append(pltpu.async_copy(rows0, accum.at[pl.ds(row0, CHUNK)], zsem))
    for cp in zcps:
        cp.wait()
    plsc.subcore_barrier()

    def _turn(g, b):
        b1 = (b - 1) % NBUF
        b3 = (b - 3) % NBUF

        @pl.when((g >= NBUF) & (g - NBUF < N_CHUNKS))
        def _():
            pltpu.make_async_copy(
                rows[b], accum.at[dmini[b]], ssems[b]).wait()

        @pl.when(g < N_CHUNKS)
        def _():
            base = pl.multiple_of(ebase + g * CHUNK, 8)
            pltpu.async_copy(src_hbm.at[pl.ds(base, CHUNK)], smini[b],
                             isems[b])
            pltpu.async_copy(dst_hbm.at[pl.ds(base, CHUNK)], dmini[b],
                             isems[b])

        g1 = g - 1

        @pl.when((g1 >= 0) & (g1 < N_CHUNKS))
        def _():
            pltpu.make_async_copy(
                src_hbm.at[pl.ds(ebase, CHUNK)], smini[b1], isems[b1]).wait()
            pltpu.make_async_copy(
                dst_hbm.at[pl.ds(ebase, CHUNK)], dmini[b1], isems[b1]).wait()
            pltpu.async_copy(r_hbm.at[smini[b1]], rows[b1], gsems[b1])

        g3 = g - 3

        @pl.when((g3 >= 0) & (g3 < N_CHUNKS))
        def _():
            pltpu.make_async_copy(
                r_hbm.at[smini[b3]], rows[b3], gsems[b3]).wait()
            pltpu.async_copy(rows[b3], accum.at[dmini[b3]], ssems[b3],
                             add=True)

    def _outer(o, carry):
        for b in range(NBUF):
            _turn(o * NBUF + b, b)
        return carry

    n_turns = N_CHUNKS + 3 + NBUF
    lax.fori_loop(0, n_turns // NBUF, _outer, 0)
    plsc.subcore_barrier()

    row0 = s * ROWS_PER_SUB
    pltpu.sync_copy(accum.at[pl.ds(row0, ROWS_PER_SUB)],
                    out_hbm.at[pl.ds(c * NPAD + row0, ROWS_PER_SUB)])


@functools.cache
def _get_sc_scatter():
    return pl.kernel(
        _sc_scatter_body,
        out_type=jax.ShapeDtypeStruct((2 * NPAD, D), _f32),
        mesh=plsc.VectorSubcoreMesh(core_axis_name="c", subcore_axis_name="s"),
        scratch_types=[
            pltpu.VMEM_SHARED((NPAD, D), _f32),
            *[pltpu.VMEM((CHUNK,), jnp.int32) for _ in range(2 * NBUF)],
            *[pltpu.VMEM((CHUNK, D), _f32) for _ in range(NBUF)],
            *[pltpu.SemaphoreType.DMA for _ in range(3 * NBUF + 1)],
        ],
    )


@jax.jit
def kernel(feat, edge_index, Wg, bg):
    src = edge_index[0]
    dst = edge_index[1]
    sc_scatter = _get_sc_scatter()
    feats = []
    r = _tc_gate(feat, Wg[0], bg[0].reshape(1, 1))
    for i in range(K):
        parts = sc_scatter(r, src, dst)
        if i < K - 1:
            h, r = _tc_blend_gate(parts, feat, Wg[i + 1], bg[i + 1].reshape(1, 1))
        else:
            h = _tc_blend(parts, feat)
        feats.append(h)
    return jnp.stack(feats, axis=0)
